# baseline (speedup 1.0000x reference)
import functools

import jax
import jax.numpy as jnp
from jax import lax
from jax.experimental import pallas as pl
from jax.experimental.pallas import tpu as pltpu
from jax.experimental.pallas import tpu_sc as plsc

TOP_K = 2


def _pick_chunk(rows, cap):
    for c in range(min(rows, cap), 0, -1):
        if rows % c == 0 and c % 8 == 0:
            return c
    return rows


def _sc_gather(table, idx, *, chunk):
    B = idx.shape[0]
    D = table.shape[1]
    info = plsc.get_sparse_core_info()
    nw = info.num_cores * info.num_subcores
    rows_per_w = B // nw
    chunk = _pick_chunk(rows_per_w, chunk)
    nchunk = rows_per_w // chunk
    mesh = plsc.VectorSubcoreMesh(core_axis_name="c", subcore_axis_name="s")

    @functools.partial(
        pl.kernel, mesh=mesh,
        out_type=jax.ShapeDtypeStruct((B, D), table.dtype),
        scratch_types=[
            pltpu.VMEM((chunk,), jnp.int32),
            pltpu.VMEM((chunk, D), table.dtype),
            pltpu.SemaphoreType.DMA,
        ],
    )
    def k(table_hbm, idx_hbm, out_hbm, idx_v, rows_v, sem):
        wid = lax.axis_index("s") * info.num_cores + lax.axis_index("c")
        base = wid * rows_per_w
        for c in range(nchunk):
            off = base + c * chunk
            pltpu.sync_copy(idx_hbm.at[pl.ds(off, chunk)], idx_v)
            pltpu.async_copy(table_hbm.at[idx_v], rows_v, sem).wait()
            pltpu.sync_copy(rows_v, out_hbm.at[pl.ds(off, chunk)])

    return k(table, idx)


def _sc_combine(ys, y_sh, pos_a, pos_b, *, chunk):
    T, D = y_sh.shape
    info = plsc.get_sparse_core_info()
    nw = info.num_cores * info.num_subcores
    rows_per_w = T // nw
    chunk = _pick_chunk(rows_per_w, chunk)
    nchunk = rows_per_w // chunk
    nl = D // 16
    nv = chunk * nl
    mesh = plsc.VectorSubcoreMesh(core_axis_name="c", subcore_axis_name="s")

    @functools.partial(
        pl.kernel, mesh=mesh,
        out_type=jax.ShapeDtypeStruct((T, D), jnp.float32),
        scratch_types=[
            pltpu.VMEM((chunk,), jnp.int32),
            pltpu.VMEM((chunk,), jnp.int32),
            pltpu.VMEM((chunk, D), jnp.float32),
            pltpu.VMEM((chunk, D), jnp.float32),
            pltpu.VMEM((chunk, D), jnp.float32),
            pltpu.SemaphoreType.DMA,
        ],
    )
    def k(ys_hbm, ysh_hbm, pa_hbm, pb_hbm, out_hbm, pa_v, pb_v, ra_v, rb_v,
          rs_v, sem):
        wid = lax.axis_index("s") * info.num_cores + lax.axis_index("c")
        base = wid * rows_per_w
        for c in range(nchunk):
            off = base + c * chunk
            pltpu.sync_copy(pa_hbm.at[pl.ds(off, chunk)], pa_v)
            pltpu.sync_copy(pb_hbm.at[pl.ds(off, chunk)], pb_v)
            pltpu.sync_copy(ysh_hbm.at[pl.ds(off, chunk)], rs_v)
            ca = pltpu.async_copy(ys_hbm.at[pa_v], ra_v, sem)
            cb = pltpu.async_copy(ys_hbm.at[pb_v], rb_v, sem)
            ca.wait()
            cb.wait()
            def body(i, _):
                r = i // nl
                l = i - r * nl
                sl = pl.ds(l * 16, 16)
                rs_v[r, sl] = rs_v[r, sl] + ra_v[r, sl] + rb_v[r, sl]
                return 0

            lax.fori_loop(0, nv, body, 0)
            pltpu.sync_copy(rs_v, out_hbm.at[pl.ds(off, chunk)])

    return k(ys, y_sh, pos_a, pos_b)


def _gate_up_kernel(gid_ref, act_ref, xs_ref, ws_ref, wg_ref, wu_ref, h_ref):
    t = pl.program_id(0)

    @pl.when(act_ref[t] == 1)
    def _():
        x = xs_ref[...]
        g = jax.lax.dot_general(x, wg_ref[0], (((1,), (1,)), ((), ())),
                                preferred_element_type=jnp.float32)
        u = jax.lax.dot_general(x, wu_ref[0], (((1,), (1,)), ((), ())),
                                preferred_element_type=jnp.float32)
        h = g * jax.nn.sigmoid(g) * u * ws_ref[0]
        h_ref[...] = h.astype(jnp.bfloat16)


def _down_kernel(gid_ref, act_ref, h_ref, wd_ref, out_ref):
    t = pl.program_id(0)

    @pl.when(act_ref[t] == 1)
    def _():
        h = h_ref[...]
        wd = wd_ref[0].astype(jnp.bfloat16)
        out_ref[...] = jax.lax.dot_general(
            h, wd, (((1,), (1,)), ((), ())),
            preferred_element_type=jnp.float32)


def _shared_ffn_kernel(x_ref, wg_ref, wu_ref, wd_ref, out_ref):
    x = x_ref[...].astype(jnp.bfloat16)
    g = jax.lax.dot_general(x, wg_ref[...], (((1,), (1,)), ((), ())),
                            preferred_element_type=jnp.float32)
    u = jax.lax.dot_general(x, wu_ref[...], (((1,), (1,)), ((), ())),
                            preferred_element_type=jnp.float32)
    h = (g * jax.nn.sigmoid(g) * u).astype(jnp.bfloat16)
    out_ref[...] = jax.lax.dot_general(h, wd_ref[...], (((1,), (1,)), ((), ())),
                                       preferred_element_type=jnp.float32)


def kernel(x, Wgate, Wg, Wu, Wd, Wg_s, Wu_s, Wd_s):
    bsz, seq_len, H = x.shape
    E, I, _ = Wg.shape
    T = bsz * seq_len
    TM = 128

    xf = x.reshape(T, H)

    logits = xf @ Wgate.T
    scores = jax.nn.softmax(logits, axis=-1)
    eiota = jnp.arange(E, dtype=jnp.int32)[None, :]
    m1 = jnp.max(scores, axis=-1)
    i1 = jnp.argmax(scores, axis=-1).astype(jnp.int32)
    s2 = jnp.where(eiota == i1[:, None], -jnp.inf, scores)
    m2 = jnp.max(s2, axis=-1)
    i2 = jnp.argmax(s2, axis=-1).astype(jnp.int32)
    denom = m1 + m2 + 1e-20
    topk_w = jnp.stack([m1 / denom, m2 / denom], axis=-1)
    topk_idx = jnp.stack([i1, i2], axis=-1)

    R = T * TOP_K
    e_pairs = topk_idx.reshape(R)
    w_pairs = topk_w.reshape(R)
    tok_pairs = jax.lax.broadcasted_iota(jnp.int32, (T, TOP_K), 0).reshape(R)

    oh = (e_pairs[:, None] == eiota)
    csum = jnp.cumsum(oh.astype(jnp.int32), axis=0)
    counts = csum[-1]
    rank = jnp.take_along_axis(csum, e_pairs[:, None], axis=1)[:, 0] - 1
    pcounts = ((counts + TM - 1) // TM) * TM
    pcum = jnp.cumsum(pcounts)
    pstarts = pcum - pcounts
    dst = pstarts[e_pairs] + rank

    P = ((R + E * (TM - 1)) + 255) // 256 * 256
    ntiles = P // TM

    gather_src = jnp.zeros((P,), jnp.int32).at[dst].set(tok_pairs)
    ws_pad = jnp.zeros((P,), jnp.float32).at[dst].set(w_pairs)
    ws_tiles = ws_pad.reshape(ntiles, TM, 1)

    tile_base = jnp.arange(ntiles, dtype=jnp.int32) * TM
    gid = jnp.sum((tile_base[:, None] >= pcum[None, :]).astype(jnp.int32),
                  axis=1)
    act = (tile_base < pcum[-1]).astype(jnp.int32)
    gid = jnp.clip(gid, 0, E - 1).astype(jnp.int32)

    xs = _sc_gather(xf, gather_src, chunk=80)

    hmat = pl.pallas_call(
        _gate_up_kernel,
        grid_spec=pltpu.PrefetchScalarGridSpec(
            num_scalar_prefetch=2,
            grid=(ntiles,),
            in_specs=[
                pl.BlockSpec((TM, H), lambda t, g_r, a_r: (t, 0)),
                pl.BlockSpec((1, TM, 1), lambda t, g_r, a_r: (t, 0, 0)),
                pl.BlockSpec((1, I, H), lambda t, g_r, a_r: (g_r[t], 0, 0)),
                pl.BlockSpec((1, I, H), lambda t, g_r, a_r: (g_r[t], 0, 0)),
            ],
            out_specs=pl.BlockSpec((TM, I), lambda t, g_r, a_r: (t, 0)),
        ),
        out_shape=jax.ShapeDtypeStruct((P, I), jnp.bfloat16),
    )(gid, act, xs, ws_tiles, Wg, Wu)

    ys = pl.pallas_call(
        _down_kernel,
        grid_spec=pltpu.PrefetchScalarGridSpec(
            num_scalar_prefetch=2,
            grid=(ntiles,),
            in_specs=[
                pl.BlockSpec((TM, I), lambda t, g_r, a_r: (t, 0)),
                pl.BlockSpec((1, H, I), lambda t, g_r, a_r: (g_r[t], 0, 0)),
            ],
            out_specs=pl.BlockSpec((TM, H), lambda t, g_r, a_r: (t, 0)),
        ),
        out_shape=jax.ShapeDtypeStruct((P, H), jnp.float32),
    )(gid, act, hmat, Wd)

    TMS = 512
    y_sh = pl.pallas_call(
        _shared_ffn_kernel,
        grid=(T // TMS,),
        in_specs=[
            pl.BlockSpec((TMS, H), lambda t: (t, 0)),
            pl.BlockSpec((I, H), lambda t: (0, 0)),
            pl.BlockSpec((I, H), lambda t: (0, 0)),
            pl.BlockSpec((H, I), lambda t: (0, 0)),
        ],
        out_specs=pl.BlockSpec((TMS, H), lambda t: (t, 0)),
        out_shape=jax.ShapeDtypeStruct((T, H), jnp.float32),
    )(xf, Wg_s.astype(jnp.bfloat16), Wu_s.astype(jnp.bfloat16),
      Wd_s.astype(jnp.bfloat16))

    pos = dst.reshape(T, TOP_K)
    y = _sc_combine(ys, y_sh, pos[:, 0], pos[:, 1], chunk=32)
    return y.reshape(bsz, seq_len, H)
